# baseline (speedup 1.0000x reference)
import functools

import jax
import jax.numpy as jnp
from jax import lax
from jax.experimental import pallas as pl
from jax.experimental.pallas import tpu as pltpu
from jax.experimental.pallas import tpu_sc as plsc

MAX_NUM_TILES = 4
HIDDEN = 1280
NUM_PATCHES = 1601
PBLK = 544


def _sc_gather_body(ar_hbm, ti_hbm, table_hbm, out_hbm, ar_v, ti_v, idx_v,
                    rows_v, sem):
    c = lax.axis_index("c")
    s = lax.axis_index("s")
    num_c = lax.axis_size("c")
    wid = s * num_c + c

    @pl.when(wid < 2)
    def _():
        pltpu.sync_copy(ar_hbm, ar_v)
        pltpu.sync_copy(ti_hbm, ti_v)
        base = wid * 16
        ar16 = ar_v[pl.ds(base, 16)]
        ti16 = ti_v[pl.ds(base, 16)]
        idx_v[...] = ar16 * MAX_NUM_TILES + ti16
        pltpu.async_copy(table_hbm.at[idx_v], rows_v, sem).wait()
        pltpu.sync_copy(rows_v, out_hbm.at[pl.ds(base, 16)])


def _sc_gather(ar, ti, table_rows):
    b = ar.shape[0]
    mesh = plsc.VectorSubcoreMesh(core_axis_name="c", subcore_axis_name="s")
    return pl.kernel(
        _sc_gather_body,
        out_type=jax.ShapeDtypeStruct((b, HIDDEN), jnp.float32),
        mesh=mesh,
        scratch_types=[
            pltpu.VMEM((b,), jnp.int32),
            pltpu.VMEM((b,), jnp.int32),
            pltpu.VMEM((16,), jnp.int32),
            pltpu.VMEM((16, HIDDEN), jnp.float32),
            pltpu.SemaphoreType.DMA,
        ],
    )(ar, ti, table_rows)


def _add_body(x_ref, emb_ref, gate_ref, o_ref):
    scale = jnp.tanh(gate_ref[...])
    add = emb_ref[...] * scale
    o_ref[...] = x_ref[...] + add


def _tc_add(x, emb, gate2):
    b = x.shape[0]
    npb = pl.cdiv(NUM_PATCHES, PBLK)
    return pl.pallas_call(
        _add_body,
        grid=(b, npb),
        in_specs=[
            pl.BlockSpec((1, PBLK, HIDDEN), lambda i, p: (i, p, 0)),
            pl.BlockSpec((1, 1, HIDDEN), lambda i, p: (i, 0, 0)),
            pl.BlockSpec((1, 1), lambda i, p: (0, 0)),
        ],
        out_specs=pl.BlockSpec((1, PBLK, HIDDEN), lambda i, p: (i, p, 0)),
        out_shape=jax.ShapeDtypeStruct(x.shape, x.dtype),
        compiler_params=pltpu.CompilerParams(
            dimension_semantics=("parallel", "parallel")),
    )(x, emb, gate2)


@jax.jit
def kernel(x, aspect_ratio_ids, tile_indices, table, gate):
    table_rows = table.reshape(-1, HIDDEN)
    emb = _sc_gather(aspect_ratio_ids.astype(jnp.int32),
                     tile_indices.astype(jnp.int32), table_rows)
    return _tc_add(x, emb.reshape(x.shape[0], 1, HIDDEN), gate.reshape(1, 1))


# speedup vs baseline: 1.0261x; 1.0261x over previous
import jax
import jax.numpy as jnp
from jax.experimental import pallas as pl
from jax.experimental.pallas import tpu as pltpu

MAX_NUM_TILES = 4
HIDDEN = 1280
NUM_PATCHES = 1601
PBLK = 544


def _add_body(rows_ref, x_ref, emb_ref, gate_ref, o_ref):
    scale = jnp.tanh(gate_ref[...])
    o_ref[...] = x_ref[...] + emb_ref[...] * scale


@jax.jit
def kernel(x, aspect_ratio_ids, tile_indices, table, gate):
    b = x.shape[0]
    npb = pl.cdiv(NUM_PATCHES, PBLK)
    table3 = table.reshape(-1, 1, HIDDEN)
    rows = (aspect_ratio_ids * MAX_NUM_TILES + tile_indices).astype(jnp.int32)
    grid_spec = pltpu.PrefetchScalarGridSpec(
        num_scalar_prefetch=1,
        grid=(b, npb),
        in_specs=[
            pl.BlockSpec((1, PBLK, HIDDEN), lambda i, p, r: (i, p, 0)),
            pl.BlockSpec((1, 1, HIDDEN), lambda i, p, r: (r[i], 0, 0)),
            pl.BlockSpec((1, 1), lambda i, p, r: (0, 0)),
        ],
        out_specs=pl.BlockSpec((1, PBLK, HIDDEN), lambda i, p, r: (i, p, 0)),
    )
    return pl.pallas_call(
        _add_body,
        grid_spec=grid_spec,
        out_shape=jax.ShapeDtypeStruct(x.shape, x.dtype),
        compiler_params=pltpu.CompilerParams(
            dimension_semantics=("parallel", "parallel")),
    )(rows, x, table3, gate.reshape(1, 1))


# speedup vs baseline: 1.0349x; 1.0087x over previous
import jax
import jax.numpy as jnp
from jax.experimental import pallas as pl
from jax.experimental.pallas import tpu as pltpu

MAX_NUM_TILES = 4
HIDDEN = 1280
NUM_PATCHES = 1601
PBLK = 1601


def _add_body(rows_ref, x_ref, emb_ref, gate_ref, o_ref):
    scale = jnp.tanh(gate_ref[...])
    o_ref[...] = x_ref[...] + emb_ref[...] * scale


@jax.jit
def kernel(x, aspect_ratio_ids, tile_indices, table, gate):
    b = x.shape[0]
    npb = pl.cdiv(NUM_PATCHES, PBLK)
    table3 = table.reshape(-1, 1, HIDDEN)
    rows = (aspect_ratio_ids * MAX_NUM_TILES + tile_indices).astype(jnp.int32)
    grid_spec = pltpu.PrefetchScalarGridSpec(
        num_scalar_prefetch=1,
        grid=(b, npb),
        in_specs=[
            pl.BlockSpec((1, PBLK, HIDDEN), lambda i, p, r: (i, p, 0)),
            pl.BlockSpec((1, 1, HIDDEN), lambda i, p, r: (r[i], 0, 0)),
            pl.BlockSpec((1, 1), lambda i, p, r: (0, 0)),
        ],
        out_specs=pl.BlockSpec((1, PBLK, HIDDEN), lambda i, p, r: (i, p, 0)),
    )
    return pl.pallas_call(
        _add_body,
        grid_spec=grid_spec,
        out_shape=jax.ShapeDtypeStruct(x.shape, x.dtype),
        compiler_params=pltpu.CompilerParams(
            dimension_semantics=("parallel", "parallel")),
    )(rows, x, table3, gate.reshape(1, 1))
